# baseline (speedup 1.0000x reference)
import functools

import jax
import jax.numpy as jnp
from jax import lax
from jax.experimental import pallas as pl
from jax.experimental.pallas import tpu as pltpu
from jax.experimental.pallas import tpu_sc as plsc

_FEATURES = 32
_NC = 2
_NS = 16
_NW = _NC * _NS
_CH = 128
_SUPER = 5
_ROWS_PER_SUPER = _CH * _SUPER


def _embed_lookup(n_flat):
    n_per_w = n_flat // _NW
    n_chunks = n_per_w // _CH
    n_super = n_chunks // _SUPER
    mesh = plsc.VectorSubcoreMesh(core_axis_name="c", subcore_axis_name="s")

    @functools.partial(
        pl.kernel,
        out_type=jax.ShapeDtypeStruct((n_flat, _FEATURES), jnp.float32),
        mesh=mesh,
        scratch_types=[
            pltpu.VMEM((n_chunks, _CH), jnp.int32),
            pltpu.VMEM((_ROWS_PER_SUPER, _FEATURES), jnp.float32),
            pltpu.SemaphoreType.DMA,
        ],
        compiler_params=pltpu.CompilerParams(use_tc_tiling_on_sc=False),
    )
    def body(idx_hbm, table_hbm, out_hbm, idx_v, rows_v, sem):
        wid = lax.axis_index("s") * _NC + lax.axis_index("c")
        base = wid * n_per_w
        pltpu.sync_copy(idx_hbm.at[wid], idx_v)

        def step(s, carry):
            copies = []
            for k in range(_SUPER):
                c = s * _SUPER + k
                copies.append(pltpu.async_copy(
                    table_hbm.at[idx_v.at[c]],
                    rows_v.at[pl.ds(k * _CH, _CH)],
                    sem))
            for cp in copies:
                cp.wait()
            pltpu.sync_copy(
                rows_v,
                out_hbm.at[pl.ds(base + s * _ROWS_PER_SUPER, _ROWS_PER_SUPER)])
            return carry

        lax.fori_loop(0, n_super, step, 0)

    return body


def kernel(inputs, embedding):
    b, s = inputs.shape
    n_flat = b * s
    idx3 = inputs.reshape(_NW, n_flat // _NW // _CH, _CH)
    out = _embed_lookup(n_flat)(idx3, embedding)
    return out.reshape(b, s, _FEATURES)
